# baseline (speedup 1.0000x reference)
import functools

import jax
import jax.numpy as jnp
from jax import lax
from jax.experimental import pallas as pl
from jax.experimental.pallas import tpu as pltpu
from jax.experimental.pallas import tpu_sc as plsc

N = 10000
E = 320000
D = 128
NP = 10240

NC = 2
NS = 16
NW = NC * NS
EPW = E // NW
B = 128
CHT = 80
PADR = CHT + 16
RPT = NP // NS

def _sc_body(with_deg, x_hbm, src_hbm, dst_hbm, z2_hbm, z1_hbm,
             out_hbm, deg_hbm, srcA, dstA, srcB, dstB, r0_v, r1_v, ones_v,
             acc_s, deg_s, sem0, sem1, semA, semB):
    c = lax.axis_index("c")
    s = lax.axis_index("s")
    wid = s * NC + c
    row0 = s * RPT
    sw = src_hbm.at[wid]
    dw = dst_hbm.at[wid]

    pltpu.sync_copy(z2_hbm, acc_s.at[pl.ds(row0, RPT)])
    if with_deg:
        pltpu.sync_copy(z1_hbm, deg_s.at[pl.ds(row0, RPT)])
        for i in range(B // 16):
            ones_v[pl.ds(i * 16, 16)] = jnp.ones((16,), jnp.float32)
    plsc.subcore_barrier()

    def scatter(dref, rows):
        pltpu.sync_copy(rows, acc_s.at[dref], add=True)
        if with_deg:
            pltpu.sync_copy(ones_v, deg_s.at[dref], add=True)

    pltpu.async_copy(sw.at[pl.ds(0, 8)], srcA, semA)
    pltpu.async_copy(dw.at[pl.ds(0, 8)], dstA, semA)
    pltpu.make_async_copy(sw.at[pl.ds(0, 8)], srcA, semA).wait()
    pltpu.make_async_copy(dw.at[pl.ds(0, 8)], dstA, semA).wait()
    pltpu.async_copy(sw.at[pl.ds(8, 8)], srcB, semB)
    pltpu.async_copy(dw.at[pl.ds(8, 8)], dstB, semB)
    pltpu.async_copy(x_hbm.at[srcA.at[0]], r0_v, sem0)
    pltpu.async_copy(x_hbm.at[srcA.at[1]], r1_v, sem1)

    def iter16(i, carry):
        j = i * 16
        halves = ((srcA, dstA, srcB, dstB, semB, semA),
                  (srcB, dstB, srcA, dstA, semA, semB))
        for h, (sc, dc, sn, dn, semn, semc) in enumerate(halves):
            base = j + 8 * h
            for t in range(8):
                r = r0_v if t % 2 == 0 else r1_v
                sem = sem0 if t % 2 == 0 else sem1
                pltpu.make_async_copy(x_hbm.at[sc.at[t]], r, sem).wait()
                scatter(dc.at[t], r)
                if t < 6:
                    pltpu.async_copy(x_hbm.at[sc.at[t + 2]], r, sem)
                elif t == 6:
                    pltpu.make_async_copy(sw.at[pl.ds(0, 8)], sn, semn).wait()
                    pltpu.make_async_copy(dw.at[pl.ds(0, 8)], dn, semn).wait()
                    pltpu.async_copy(x_hbm.at[sn.at[0]], r, sem)
                else:
                    pltpu.async_copy(x_hbm.at[sn.at[1]], r, sem)
                    nxt = base + 16
                    pltpu.async_copy(sw.at[pl.ds(nxt, 8)], sc, semc)
                    pltpu.async_copy(dw.at[pl.ds(nxt, 8)], dc, semc)
        return carry

    lax.fori_loop(0, CHT // 16, iter16, 0)
    pltpu.make_async_copy(x_hbm.at[srcA.at[0]], r0_v, sem0).wait()
    pltpu.make_async_copy(x_hbm.at[srcA.at[1]], r1_v, sem1).wait()
    pltpu.make_async_copy(sw.at[pl.ds(0, 8)], srcB, semB).wait()
    pltpu.make_async_copy(dw.at[pl.ds(0, 8)], dstB, semB).wait()
    plsc.subcore_barrier()

    out0 = c * NP + row0
    pltpu.sync_copy(acc_s.at[pl.ds(row0, RPT)], out_hbm.at[pl.ds(out0, RPT)])
    if with_deg:
        pltpu.sync_copy(deg_s.at[pl.ds(row0, RPT)], deg_hbm.at[pl.ds(out0, RPT)])


@functools.lru_cache(maxsize=None)
def _make_sc_pass(with_deg):
    mesh = plsc.VectorSubcoreMesh(core_axis_name="c", subcore_axis_name="s")
    out_type = [jax.ShapeDtypeStruct((NC * NP, D), jnp.float32)]
    if with_deg:
        out_type.append(jax.ShapeDtypeStruct((NC * NP,), jnp.float32))
    kern = functools.partial(
        pl.kernel,
        mesh=mesh,
        out_type=out_type,
        scratch_types=[
            pltpu.VMEM((8, B), jnp.int32),
            pltpu.VMEM((8, B), jnp.int32),
            pltpu.VMEM((8, B), jnp.int32),
            pltpu.VMEM((8, B), jnp.int32),
            pltpu.VMEM((B, D), jnp.float32),
            pltpu.VMEM((B, D), jnp.float32),
            pltpu.VMEM((B,), jnp.float32),
            pltpu.VMEM_SHARED((NP, D), jnp.float32),
            pltpu.VMEM_SHARED((NP,), jnp.float32),
            pltpu.SemaphoreType.DMA,
            pltpu.SemaphoreType.DMA,
            pltpu.SemaphoreType.DMA,
            pltpu.SemaphoreType.DMA,
        ],
    )

    if with_deg:
        @kern
        def sc_pass(x_hbm, src_hbm, dst_hbm, z2_hbm, z1_hbm, out_hbm, deg_hbm,
                    *scratch):
            _sc_body(True, x_hbm, src_hbm, dst_hbm, z2_hbm, z1_hbm,
                     out_hbm, deg_hbm, *scratch)
    else:
        @kern
        def sc_pass(x_hbm, src_hbm, dst_hbm, z2_hbm, out_hbm, *scratch):
            _sc_body(False, x_hbm, src_hbm, dst_hbm, z2_hbm, None,
                     out_hbm, None, *scratch)

    return sc_pass

BR = 1024
NB = NP // BR


def _dense_body(sa, sb, da, db, x, wl, wr, b, o, *, relu):
    deg = jnp.maximum(da[...] + db[...], 1.0)
    agg = (sa[...] + sb[...]) * (1.0 / deg)[:, None]
    y = jnp.dot(agg, wl[...], preferred_element_type=jnp.float32)
    y = y + jnp.dot(x[...], wr[...], preferred_element_type=jnp.float32)
    y = y + b[...]
    o[...] = jnp.maximum(y, 0.0) if relu else y


def _dense(summed, deg, xin, WlT, WrT, b, relu):
    return pl.pallas_call(
        functools.partial(_dense_body, relu=relu),
        grid=(NB,),
        in_specs=[
            pl.BlockSpec((BR, D), lambda i: (i, 0)),
            pl.BlockSpec((BR, D), lambda i: (i + NB, 0)),
            pl.BlockSpec((BR,), lambda i: (i,)),
            pl.BlockSpec((BR,), lambda i: (i + NB,)),
            pl.BlockSpec((BR, D), lambda i: (i, 0)),
            pl.BlockSpec((D, D), lambda i: (0, 0)),
            pl.BlockSpec((D, D), lambda i: (0, 0)),
            pl.BlockSpec((1, D), lambda i: (0, 0)),
        ],
        out_specs=pl.BlockSpec((BR, D), lambda i: (i, 0)),
        out_shape=jax.ShapeDtypeStruct((NP, D), jnp.float32),
    )(summed, summed, deg, deg, xin, WlT, WrT, b)


def kernel(x, edge_index, W1_l, b1, W1_r, W2_l, b2, W2_r):
    src = jnp.pad(edge_index[0].astype(jnp.int32).reshape(NW, EPW),
                  ((0, 0), (0, PADR * B - EPW))).reshape(NW, PADR, B)
    dst = jnp.pad(edge_index[1].astype(jnp.int32).reshape(NW, EPW),
                  ((0, 0), (0, PADR * B - EPW)),
                  constant_values=NP - 1).reshape(NW, PADR, B)
    z2 = jnp.zeros((RPT, D), jnp.float32)
    z1 = jnp.zeros((RPT,), jnp.float32)
    x_pad = jnp.pad(x, ((0, NP - N), (0, 0)))

    summed1, deg = _make_sc_pass(True)(x, src, dst, z2, z1)
    h = _dense(summed1, deg, x_pad, W1_l.T, W1_r.T, b1.reshape(1, D),
               relu=True)
    (summed2,) = _make_sc_pass(False)(h, src, dst, z2)
    out = _dense(summed2, deg, h, W2_l.T, W2_r.T, b2.reshape(1, D),
                 relu=False)
    return out[:N]


# speedup vs baseline: 1.6724x; 1.6724x over previous
import functools

import jax
import jax.numpy as jnp
from jax import lax
from jax.experimental import pallas as pl
from jax.experimental.pallas import tpu as pltpu
from jax.experimental.pallas import tpu_sc as plsc

N = 10000
E = 320000
D = 128
NP = 10240

NC = 2
NS = 16
NW = NC * NS
EPW = E // NW
B = 80
CH = EPW // B
CHE = 126
RING = 32
PADR = 152
RPT = NP // NS

def _sc_body(with_deg, x_hbm, src_hbm, dst_hbm, z2_hbm, z1_hbm,
             out_hbm, deg_hbm, ring_v, dst_v, r0_v, r1_v, ones_v,
             acc_s, deg_s, sem0, sem1, semR):
    c = lax.axis_index("c")
    s = lax.axis_index("s")
    wid = s * NC + c
    row0 = s * RPT
    sw = src_hbm.at[wid]
    dw = dst_hbm.at[wid]

    pltpu.sync_copy(z2_hbm, acc_s.at[pl.ds(row0, RPT)])
    if with_deg:
        pltpu.sync_copy(z1_hbm, deg_s.at[pl.ds(row0, RPT)])
        for i in range(B // 16):
            ones_v[pl.ds(i * 16, 16)] = jnp.ones((16,), jnp.float32)
    plsc.subcore_barrier()

    pltpu.sync_copy(dw.at[pl.ds(0, 128)], dst_v)

    def scatter(j, rows):
        pltpu.sync_copy(rows, acc_s.at[dst_v.at[j]], add=True)
        if with_deg:
            pltpu.sync_copy(ones_v, deg_s.at[dst_v.at[j]], add=True)

    pltpu.sync_copy(sw.at[pl.ds(0, 16)], ring_v.at[pl.ds(0, 16)])
    pltpu.async_copy(sw.at[pl.ds(16, 8)], ring_v.at[pl.ds(16, 8)], semR)
    pltpu.async_copy(x_hbm.at[ring_v.at[0]], r0_v, sem0)
    pltpu.async_copy(x_hbm.at[ring_v.at[1]], r1_v, sem1)

    def pair(p, carry):
        j = p * 2

        @pl.when(lax.rem(j, 8) == 6)
        def _():
            k = pl.multiple_of(j + 18, 8)
            slot = pl.multiple_of(lax.rem(j + 18, RING), 8)
            pltpu.make_async_copy(sw.at[pl.ds(0, 8)],
                                  ring_v.at[pl.ds(0, 8)], semR).wait()
            pltpu.async_copy(sw.at[pl.ds(k, 8)],
                             ring_v.at[pl.ds(slot, 8)], semR)

        pltpu.make_async_copy(x_hbm.at[ring_v.at[0]], r0_v, sem0).wait()
        scatter(j, r0_v)
        pltpu.async_copy(x_hbm.at[ring_v.at[lax.rem(j + 2, RING)]],
                         r0_v, sem0)
        pltpu.make_async_copy(x_hbm.at[ring_v.at[1]], r1_v, sem1).wait()
        scatter(j + 1, r1_v)
        pltpu.async_copy(x_hbm.at[ring_v.at[lax.rem(j + 3, RING)]],
                         r1_v, sem1)
        return carry

    lax.fori_loop(0, CHE // 2, pair, 0)
    pltpu.make_async_copy(x_hbm.at[ring_v.at[0]], r0_v, sem0).wait()
    pltpu.make_async_copy(x_hbm.at[ring_v.at[1]], r1_v, sem1).wait()
    pltpu.make_async_copy(sw.at[pl.ds(0, 8)],
                          ring_v.at[pl.ds(0, 8)], semR).wait()
    plsc.subcore_barrier()

    out0 = c * NP + row0
    pltpu.sync_copy(acc_s.at[pl.ds(row0, RPT)], out_hbm.at[pl.ds(out0, RPT)])
    if with_deg:
        pltpu.sync_copy(deg_s.at[pl.ds(row0, RPT)], deg_hbm.at[pl.ds(out0, RPT)])


@functools.lru_cache(maxsize=None)
def _make_sc_pass(with_deg):
    mesh = plsc.VectorSubcoreMesh(core_axis_name="c", subcore_axis_name="s")
    out_type = [jax.ShapeDtypeStruct((NC * NP, D), jnp.float32)]
    if with_deg:
        out_type.append(jax.ShapeDtypeStruct((NC * NP,), jnp.float32))
    kern = functools.partial(
        pl.kernel,
        mesh=mesh,
        out_type=out_type,
        scratch_types=[
            pltpu.VMEM((RING, B), jnp.int32),
            pltpu.VMEM((128, B), jnp.int32),
            pltpu.VMEM((B, D), jnp.float32),
            pltpu.VMEM((B, D), jnp.float32),
            pltpu.VMEM((B,), jnp.float32),
            pltpu.VMEM_SHARED((NP, D), jnp.float32),
            pltpu.VMEM_SHARED((NP,), jnp.float32),
            pltpu.SemaphoreType.DMA,
            pltpu.SemaphoreType.DMA,
            pltpu.SemaphoreType.DMA,
        ],
    )

    if with_deg:
        @kern
        def sc_pass(x_hbm, src_hbm, dst_hbm, z2_hbm, z1_hbm, out_hbm, deg_hbm,
                    *scratch):
            _sc_body(True, x_hbm, src_hbm, dst_hbm, z2_hbm, z1_hbm,
                     out_hbm, deg_hbm, *scratch)
    else:
        @kern
        def sc_pass(x_hbm, src_hbm, dst_hbm, z2_hbm, out_hbm, *scratch):
            _sc_body(False, x_hbm, src_hbm, dst_hbm, z2_hbm, None,
                     out_hbm, None, *scratch)

    return sc_pass

BR = 1024
NB = NP // BR


def _dense_body(sa, sb, da, db, x, wl, wr, b, o, *, relu):
    deg = jnp.maximum(da[...] + db[...], 1.0)
    agg = (sa[...] + sb[...]) * (1.0 / deg)[:, None]
    y = jnp.dot(agg, wl[...], preferred_element_type=jnp.float32)
    y = y + jnp.dot(x[...], wr[...], preferred_element_type=jnp.float32)
    y = y + b[...]
    o[...] = jnp.maximum(y, 0.0) if relu else y


def _dense(summed, deg, xin, WlT, WrT, b, relu):
    return pl.pallas_call(
        functools.partial(_dense_body, relu=relu),
        grid=(NB,),
        in_specs=[
            pl.BlockSpec((BR, D), lambda i: (i, 0)),
            pl.BlockSpec((BR, D), lambda i: (i + NB, 0)),
            pl.BlockSpec((BR,), lambda i: (i,)),
            pl.BlockSpec((BR,), lambda i: (i + NB,)),
            pl.BlockSpec((BR, D), lambda i: (i, 0)),
            pl.BlockSpec((D, D), lambda i: (0, 0)),
            pl.BlockSpec((D, D), lambda i: (0, 0)),
            pl.BlockSpec((1, D), lambda i: (0, 0)),
        ],
        out_specs=pl.BlockSpec((BR, D), lambda i: (i, 0)),
        out_shape=jax.ShapeDtypeStruct((NP, D), jnp.float32),
    )(summed, summed, deg, deg, xin, WlT, WrT, b)


def kernel(x, edge_index, W1_l, b1, W1_r, W2_l, b2, W2_r):
    src = jnp.pad(edge_index[0].astype(jnp.int32).reshape(NW, EPW),
                  ((0, 0), (0, PADR * B - EPW))).reshape(NW, PADR, B)
    dst = jnp.pad(edge_index[1].astype(jnp.int32).reshape(NW, EPW),
                  ((0, 0), (0, PADR * B - EPW)),
                  constant_values=NP - 1).reshape(NW, PADR, B)
    z2 = jnp.zeros((RPT, D), jnp.float32)
    z1 = jnp.zeros((RPT,), jnp.float32)
    x_pad = jnp.pad(x, ((0, NP - N), (0, 0)))

    summed1, deg = _make_sc_pass(True)(x, src, dst, z2, z1)
    h = _dense(summed1, deg, x_pad, W1_l.T, W1_r.T, b1.reshape(1, D),
               relu=True)
    (summed2,) = _make_sc_pass(False)(h, src, dst, z2)
    out = _dense(summed2, deg, h, W2_l.T, W2_r.T, b2.reshape(1, D),
                 relu=False)
    return out[:N]
